# baseline (speedup 1.0000x reference)
import functools

import jax
import jax.numpy as jnp
from jax import lax
from jax.experimental import pallas as pl
from jax.experimental.pallas import tpu as pltpu
from jax.experimental.pallas import tpu_sc as plsc

VOCAB = 1000
DIM = 64
PDIM = 128
BATCH = 1024
SEQ = 50

NUM_CORES = 2
NUM_SUBCORES = 16
NUM_WORKERS = NUM_CORES * NUM_SUBCORES
HALVES = 2
HBATCH = BATCH // HALVES
ROWS_PER_W = HBATCH // NUM_WORKERS
BLK = 4
NCHUNK = ROWS_PER_W // BLK
CT = BLK * SEQ


@functools.lru_cache(maxsize=1)
def _build():
    mesh = plsc.VectorSubcoreMesh(core_axis_name="c", subcore_axis_name="s")

    @functools.partial(
        pl.kernel,
        mesh=mesh,
        out_type=jax.ShapeDtypeStruct((HBATCH, SEQ, DIM), jnp.float32),
        scratch_types=[
            pltpu.VMEM_SHARED((VOCAB, PDIM), jnp.float32),
            pltpu.VMEM((ROWS_PER_W, SEQ), jnp.int32),
            pltpu.VMEM((2, CT, PDIM), jnp.float32),
            pltpu.VMEM((2, BLK, SEQ, DIM), jnp.float32),
            pltpu.SemaphoreType.DMA,
            pltpu.SemaphoreType.DMA,
        ],
    )
    def gather_kernel(
        idx_hbm, table_hbm, out_hbm, table_s, idx_v, gbuf, obuf, gsem, wsem
    ):
        sid = lax.axis_index("s")
        wid = sid * NUM_CORES + lax.axis_index("c")
        base = wid * ROWS_PER_W

        @pl.when(sid == 0)
        def _():
            pltpu.sync_copy(table_hbm, table_s)

        pltpu.sync_copy(idx_hbm.at[pl.ds(base, ROWS_PER_W)], idx_v)
        plsc.subcore_barrier()

        def fire_gathers(c, slot):
            return [
                pltpu.async_copy(
                    table_s.at[idx_v.at[c * BLK + b]],
                    gbuf.at[slot].at[pl.ds(b * SEQ, SEQ)],
                    gsem,
                )
                for b in range(BLK)
            ]

        def relay(slot):
            for b in range(BLK):

                @pl.loop(0, SEQ)
                def _(s):
                    for l in range(DIM // 16):
                        obuf[slot, b, s, pl.ds(l * 16, 16)] = gbuf[
                            slot, b * SEQ + s, pl.ds(l * 16, 16)
                        ]

        def write_block(c, slot):
            return pltpu.async_copy(
                obuf.at[slot], out_hbm.at[pl.ds(base + c * BLK, BLK)], wsem
            )

        wbs = [None, None]

        @pl.loop(0, NCHUNK // 2)
        def _(i):
            c0 = 2 * i
            g0 = fire_gathers(c0, 0)
            g1 = fire_gathers(c0 + 1, 1)
            for g in g0:
                g.wait()
            relay(0)
            wb0 = write_block(c0, 0)
            for g in g1:
                g.wait()
            relay(1)
            wb1 = write_block(c0 + 1, 1)
            wb0.wait()
            wb1.wait()

    return gather_kernel


def kernel(token_ids, W):
    wp = jnp.pad(W, ((0, 0), (0, PDIM - DIM)))
    ids = token_ids.astype(jnp.int32)
    k = _build()
    halves = [k(ids[h * HBATCH:(h + 1) * HBATCH], wp) for h in range(HALVES)]
    return jnp.concatenate(halves, axis=0)


# speedup vs baseline: 1.1582x; 1.1582x over previous
import functools

import jax
import jax.numpy as jnp
from jax import lax
from jax.experimental import pallas as pl
from jax.experimental.pallas import tpu as pltpu
from jax.experimental.pallas import tpu_sc as plsc

VOCAB = 1000
DIM = 64
PDIM = 128
BATCH = 1024
SEQ = 50

NUM_CORES = 2
NUM_SUBCORES = 16
NUM_WORKERS = NUM_CORES * NUM_SUBCORES
ROWS_PER_W = BATCH // NUM_WORKERS
BLK = 4
NCHUNK = ROWS_PER_W // BLK
CT = BLK * SEQ


@functools.lru_cache(maxsize=1)
def _build():
    mesh = plsc.VectorSubcoreMesh(core_axis_name="c", subcore_axis_name="s")

    @functools.partial(
        pl.kernel,
        mesh=mesh,
        out_type=jax.ShapeDtypeStruct((BATCH, SEQ, DIM), jnp.float32),
        scratch_types=[
            pltpu.VMEM_SHARED((VOCAB, PDIM), jnp.float32),
            pltpu.VMEM((ROWS_PER_W, SEQ), jnp.int32),
            pltpu.VMEM((2, CT, PDIM), jnp.float32),
            pltpu.VMEM((2, BLK, SEQ, DIM), jnp.float32),
            pltpu.SemaphoreType.DMA,
            pltpu.SemaphoreType.DMA,
        ],
    )
    def gather_kernel(
        idx_hbm, table_hbm, out_hbm, table_s, idx_v, gbuf, obuf, gsem, wsem
    ):
        sid = lax.axis_index("s")
        wid = sid * NUM_CORES + lax.axis_index("c")
        base = wid * ROWS_PER_W

        @pl.when(sid == 0)
        def _():
            pltpu.sync_copy(table_hbm, table_s)

        pltpu.sync_copy(idx_hbm.at[pl.ds(base, ROWS_PER_W)], idx_v)
        plsc.subcore_barrier()

        def fire_gathers(c, slot):
            return [
                pltpu.async_copy(
                    table_s.at[idx_v.at[c * BLK + b]],
                    gbuf.at[slot].at[pl.ds(b * SEQ, SEQ)],
                    gsem,
                )
                for b in range(BLK)
            ]

        def relay(slot):
            for b in range(BLK):

                @pl.loop(0, SEQ)
                def _(s):
                    for l in range(DIM // 16):
                        obuf[slot, b, s, pl.ds(l * 16, 16)] = gbuf[
                            slot, b * SEQ + s, pl.ds(l * 16, 16)
                        ]

        def write_block(c, slot):
            return pltpu.async_copy(
                obuf.at[slot], out_hbm.at[pl.ds(base + c * BLK, BLK)], wsem
            )

        wbs = [None, None]

        @pl.loop(0, NCHUNK // 2)
        def _(i):
            c0 = 2 * i
            g0 = fire_gathers(c0, 0)
            g1 = fire_gathers(c0 + 1, 1)
            for g in g0:
                g.wait()
            relay(0)
            wb0 = write_block(c0, 0)
            for g in g1:
                g.wait()
            relay(1)
            wb1 = write_block(c0 + 1, 1)
            wb0.wait()
            wb1.wait()

    return gather_kernel


def kernel(token_ids, W):
    wp = jnp.pad(W, ((0, 0), (0, PDIM - DIM)))
    return _build()(token_ids.astype(jnp.int32), wp)
